# baseline (speedup 1.0000x reference)
import functools
import jax
import jax.numpy as jnp
from jax import lax
from jax.experimental import pallas as pl
from jax.experimental.pallas import tpu as pltpu
from jax.experimental.pallas import tpu_sc as plsc

_SCALE = 8.0
_IDXW = 128
_NBUF = 4
_NCOMP = 2
_G = 128
_PADW = 128


def _scale_pad_body(t_ref, o_ref):
    t = t_ref[...]
    o_ref[...] = jnp.concatenate(
        [t * _SCALE, jnp.zeros_like(t)], axis=1)


def _scale_pad_table(table):
    v, d = table.shape
    vpad = (v + 7) // 8 * 8
    br = 1024
    grid = (vpad + br - 1) // br
    return pl.pallas_call(
        _scale_pad_body,
        out_shape=jax.ShapeDtypeStruct((vpad, 2 * d), table.dtype),
        grid=(grid,),
        in_specs=[pl.BlockSpec((br, d), lambda i: (i, 0))],
        out_specs=pl.BlockSpec((br, 2 * d), lambda i: (i, 0)),
    )(table)


@functools.cache
def _make_gather(v, d, b):
    info = plsc.get_sparse_core_info()
    nw = info.num_cores * info.num_subcores
    nc = info.num_cores
    b_per_w = b // nw
    rows_per_body = _NBUF * _G
    n_bodies = b_per_w // rows_per_body
    assert b_per_w % rows_per_body == 0
    idx_rows_per_body = _NBUF
    gh = _G // 2
    mesh = plsc.VectorSubcoreMesh(core_axis_name="c", subcore_axis_name="s")

    scratch = (
        [pltpu.VMEM((3, idx_rows_per_body, _IDXW), jnp.int32)]
        + [pltpu.VMEM((_G, _PADW), jnp.float32) for _ in range(_NBUF)]
        + [pltpu.VMEM((gh, _PADW), jnp.float32) for _ in range(_NCOMP)]
        + [pltpu.SemaphoreType.DMA for _ in range(_NBUF)]
        + [pltpu.SemaphoreType.DMA for _ in range(_NCOMP)]
        + [pltpu.SemaphoreType.DMA((3,))]
    )

    @functools.partial(
        pl.kernel,
        mesh=mesh,
        out_type=jax.ShapeDtypeStruct((b // 2, _PADW), jnp.float32),
        scratch_types=scratch,
    )
    def gather_kernel(table_hbm, idx_hbm, out_hbm, idx_v, *bufs_and_sems):
        rows = bufs_and_sems[:_NBUF]
        comp = bufs_and_sems[_NBUF:_NBUF + _NCOMP]
        sem_g = bufs_and_sems[_NBUF + _NCOMP:2 * _NBUF + _NCOMP]
        sem_w = bufs_and_sems[2 * _NBUF + _NCOMP:2 * _NBUF + 2 * _NCOMP]
        sem_i = bufs_and_sems[2 * _NBUF + 2 * _NCOMP]
        wid = lax.axis_index("s") * nc + lax.axis_index("c")
        idx_row0 = wid * (b_per_w // _IDXW)
        out_row0 = wid * (b_per_w // 2)

        def idx_chunk_copy(c, ib):
            return pltpu.make_async_copy(
                idx_hbm.at[pl.ds(idx_row0 + c * idx_rows_per_body,
                                 idx_rows_per_body)],
                idx_v.at[ib],
                sem_i.at[ib],
            )

        def write_copy(group, cc):
            return pltpu.make_async_copy(
                comp[cc],
                out_hbm.at[pl.ds(out_row0 + group * gh, gh)],
                sem_w[cc],
            )

        def repack(src, dst):
            def rp(r, carry):
                for half in range(2):
                    for q in range(d // 16):
                        dst[r, pl.ds(half * d + q * 16, 16)] = (
                            src[2 * r + half, pl.ds(q * 16, 16)])
                return carry
            lax.fori_loop(0, gh, rp, 0)

        idx_chunk_copy(0, 0).start()
        idx_chunk_copy(0, 0).wait()
        idx_chunk_copy(1, 1).start()

        def body_fn(c, carry):
            ci = lax.rem(c, 3)
            @pl.when(c >= 1)
            def _():
                idx_chunk_copy(c, ci).wait()

            @pl.when(c + 2 <= n_bodies - 1)
            def _():
                idx_chunk_copy(c + 2, lax.rem(c + 2, 3)).start()

            gathers = [
                pltpu.async_copy(
                    table_hbm.at[idx_v.at[ci, bb]],
                    rows[bb],
                    sem_g[bb],
                )
                for bb in range(_NBUF)
            ]

            for bb in range(_NBUF):
                gathers[bb].wait()
                cc = bb % _NCOMP
                if bb >= _NCOMP:
                    write_copy(c * _NBUF + bb - _NCOMP, cc).wait()
                else:
                    @pl.when(c >= 1)
                    def _(cc=cc, bb=bb):
                        write_copy((c - 1) * _NBUF + bb + _NCOMP, cc).wait()
                repack(rows[bb], comp[cc])
                write_copy(c * _NBUF + bb, cc).start()
            return carry

        lax.fori_loop(0, n_bodies, body_fn, 0)

        for bb in range(_NBUF - _NCOMP, _NBUF):
            write_copy((n_bodies - 1) * _NBUF + bb, bb % _NCOMP).wait()

    return gather_kernel


def kernel(x, table):
    v, d = table.shape
    bt, s = x.shape
    b = bt * s
    scaled = _scale_pad_table(table)
    idx2d = x.reshape(b // _IDXW, _IDXW)
    out = _make_gather(v, d, b)(scaled, idx2d)
    return out.reshape(bt, s, d)


# speedup vs baseline: 2.0678x; 2.0678x over previous
import functools
import jax
import jax.numpy as jnp
from jax import lax
from jax.experimental import pallas as pl
from jax.experimental.pallas import tpu as pltpu
from jax.experimental.pallas import tpu_sc as plsc

_SCALE = 8.0
_IDXW = 128
_NBUF = 4
_NCOMP = 2
_G = 128
_PADW = 128


def _scale_pad_body(t_ref, o_ref):
    t = t_ref[...]
    o_ref[...] = jnp.concatenate(
        [t * _SCALE, jnp.zeros_like(t)], axis=1)


def _scale_pad_table(table):
    v, d = table.shape
    br = 1024
    grid = (v + br - 1) // br
    return pl.pallas_call(
        _scale_pad_body,
        out_shape=jax.ShapeDtypeStruct((v, 2 * d), table.dtype),
        grid=(grid,),
        in_specs=[pl.BlockSpec((br, d), lambda i: (i, 0))],
        out_specs=pl.BlockSpec((br, 2 * d), lambda i: (i, 0)),
    )(table)


@functools.cache
def _make_gather(v, d, b):
    info = plsc.get_sparse_core_info()
    nw = info.num_cores * info.num_subcores
    nc = info.num_cores
    b_per_w = b // nw
    rows_per_body = _NBUF * _G
    n_bodies = b_per_w // rows_per_body
    assert b_per_w % rows_per_body == 0
    idx_rows_per_body = _NBUF
    mesh = plsc.VectorSubcoreMesh(core_axis_name="c", subcore_axis_name="s")

    scratch = (
        [pltpu.VMEM((3, idx_rows_per_body, _IDXW), jnp.int32)]
        + [pltpu.VMEM((_G, _PADW), jnp.float32) for _ in range(_NBUF)]
        + [pltpu.VMEM((_G, d), jnp.float32) for _ in range(_NCOMP)]
        + [pltpu.SemaphoreType.DMA for _ in range(_NBUF)]
        + [pltpu.SemaphoreType.DMA for _ in range(_NCOMP)]
        + [pltpu.SemaphoreType.DMA((3,))]
    )

    @functools.partial(
        pl.kernel,
        mesh=mesh,
        out_type=jax.ShapeDtypeStruct((b, d), jnp.float32),
        scratch_types=scratch,
    )
    def gather_kernel(table_hbm, idx_hbm, out_hbm, idx_v, *bufs_and_sems):
        rows = bufs_and_sems[:_NBUF]
        comp = bufs_and_sems[_NBUF:_NBUF + _NCOMP]
        sem_g = bufs_and_sems[_NBUF + _NCOMP:2 * _NBUF + _NCOMP]
        sem_w = bufs_and_sems[2 * _NBUF + _NCOMP:2 * _NBUF + 2 * _NCOMP]
        sem_i = bufs_and_sems[2 * _NBUF + 2 * _NCOMP]
        wid = lax.axis_index("s") * nc + lax.axis_index("c")
        idx_row0 = wid * (b_per_w // _IDXW)
        out_row0 = wid * b_per_w

        def idx_chunk_copy(c, ib):
            return pltpu.make_async_copy(
                idx_hbm.at[pl.ds(idx_row0 + c * idx_rows_per_body,
                                 idx_rows_per_body)],
                idx_v.at[ib],
                sem_i.at[ib],
            )

        def write_copy(group, cc):
            return pltpu.make_async_copy(
                comp[cc],
                out_hbm.at[pl.ds(out_row0 + group * _G, _G)],
                sem_w[cc],
            )

        def repack(src, dst):
            @plsc.parallel_loop(0, _G, unroll=8)
            def _(r):
                for q in range(d // 16):
                    dst[r, pl.ds(q * 16, 16)] = src[r, pl.ds(q * 16, 16)]

        idx_chunk_copy(0, 0).start()
        idx_chunk_copy(0, 0).wait()
        idx_chunk_copy(1, 1).start()

        def body_fn(c, carry):
            ci = lax.rem(c, 3)
            @pl.when(c >= 1)
            def _():
                idx_chunk_copy(c, ci).wait()

            @pl.when(c + 2 <= n_bodies - 1)
            def _():
                idx_chunk_copy(c + 2, lax.rem(c + 2, 3)).start()

            gathers = [
                pltpu.async_copy(
                    table_hbm.at[idx_v.at[ci, bb]],
                    rows[bb],
                    sem_g[bb],
                )
                for bb in range(_NBUF)
            ]

            for bb in range(_NBUF):
                gathers[bb].wait()
                cc = bb % _NCOMP
                if bb >= _NCOMP:
                    write_copy(c * _NBUF + bb - _NCOMP, cc).wait()
                else:
                    @pl.when(c >= 1)
                    def _(cc=cc, bb=bb):
                        write_copy((c - 1) * _NBUF + bb + _NCOMP, cc).wait()
                repack(rows[bb], comp[cc])
                write_copy(c * _NBUF + bb, cc).start()
            return carry

        lax.fori_loop(0, n_bodies, body_fn, 0)

        for bb in range(_NBUF - _NCOMP, _NBUF):
            write_copy((n_bodies - 1) * _NBUF + bb, bb % _NCOMP).wait()

    return gather_kernel


def kernel(x, table):
    v, d = table.shape
    bt, s = x.shape
    b = bt * s
    scaled = _scale_pad_table(table)
    idx2d = x.reshape(b // _IDXW, _IDXW)
    out = _make_gather(v, d, b)(scaled, idx2d)
    return out.reshape(bt, s, d)
